# baseline (speedup 1.0000x reference)
import random

import numpy as np
import jax
import jax.numpy as jnp
from jax import lax
from jax.experimental import pallas as pl
from jax.experimental.pallas import tpu as pltpu
from jax.experimental.pallas import tpu_sc as plsc

N = 2000
N_PAIRS = N * (N - 1) // 2
M = 32000
E = 2 * M
N_NEG_MIN = N_PAIRS - M

_rng = random.Random(0)
_pool_np = np.array([_rng.getrandbits(32) >> 11 for _ in range(131072)],
                    dtype=np.int64)
_seen = set()
_u_list = []
for _v in _pool_np:
    _v = int(_v)
    if _v in _seen:
        continue
    _seen.add(_v)
    if _v < N_PAIRS:
        _u_list.append(_v)
_U = np.array(_u_list, dtype=np.int32)
_T_MAX = int(np.searchsorted(np.cumsum(_U < N_NEG_MIN), M, side="left")) + 1
_P_CONST = _U[:_T_MAX]

try:
    _info = plsc.get_sparse_core_info()
    _NC, _NS = _info.num_cores, _info.num_subcores
except Exception:
    _NC, _NS = 2, 16
NW = _NC * _NS
EPW = E // NW
CB = 400
D = 128


def _sc_score_kernel(x_hbm, ia_hbm, ib_hbm, out_hbm,
                     ia_v, ib_v, rows_a, rows_b, scores_v, sem):
    wid = lax.axis_index("s") * _NC + lax.axis_index("c")
    lane = lax.iota(jnp.int32, 16)

    def chunk_body(c, carry):
        base = wid * EPW + c * CB
        pltpu.sync_copy(ia_hbm.at[pl.ds(base, CB)], ia_v)
        pltpu.sync_copy(ib_hbm.at[pl.ds(base, CB)], ib_v)
        pltpu.async_copy(x_hbm.at[ia_v], rows_a, sem).wait()
        pltpu.async_copy(x_hbm.at[ib_v], rows_b, sem).wait()
        def group_body(g, carry2):
            def lane_body(r, vec):
                e = g * 16 + r

                def dim_body(k, acc):
                    a = rows_a[e, pl.ds(k * 16, 16)]
                    b = rows_b[e, pl.ds(k * 16, 16)]
                    return acc + a * b

                acc = lax.fori_loop(0, D // 16, dim_body,
                                    jnp.zeros((16,), jnp.float32))
                s = jnp.sum(acc)
                return jnp.where(lane == r, s, vec)

            vec = lax.fori_loop(0, 16, lane_body, jnp.zeros((16,), jnp.float32))
            scores_v[pl.ds(g * 16, 16)] = vec
            return carry2

        lax.fori_loop(0, CB // 16, group_body, 0)
        pltpu.sync_copy(scores_v, out_hbm.at[pl.ds(base, CB)])
        return carry

    lax.fori_loop(0, EPW // CB, chunk_body, 0)


def _sc_scores(x, ia, ib):
    mesh = plsc.VectorSubcoreMesh(core_axis_name="c", subcore_axis_name="s")
    return pl.kernel(
        _sc_score_kernel,
        mesh=mesh,
        compiler_params=pltpu.CompilerParams(needs_layout_passes=False),
        out_type=jax.ShapeDtypeStruct((E,), jnp.float32),
        scratch_types=[
            pltpu.VMEM((CB,), jnp.int32),
            pltpu.VMEM((CB,), jnp.int32),
            pltpu.VMEM((CB, D), jnp.float32),
            pltpu.VMEM((CB, D), jnp.float32),
            pltpu.VMEM((CB,), jnp.float32),
            pltpu.SemaphoreType.DMA,
        ],
    )(x, ia, ib)


def _tc_loss_kernel(s_ref, o_ref):
    s = s_ref[...]
    row = lax.broadcasted_iota(jnp.int32, s.shape, 0)
    z = jnp.where(row < (M // 128), -s, s)
    sp = jnp.maximum(z, 0.0) + jnp.log1p(jnp.exp(-jnp.abs(z)))
    total = jnp.sum(sp) * (1.0 / (float(M) * float(M)))
    o_ref[...] = jnp.broadcast_to(total, (1, 1))


def _tc_loss(scores):
    s2 = scores.reshape(E // 128, 128)
    out = pl.pallas_call(
        _tc_loss_kernel,
        out_shape=jax.ShapeDtypeStruct((1, 1), jnp.float32),
    )(s2)
    return out[0, 0]


def _tri_offset(i):
    return i * (N - 1) - (i * (i - 1)) // 2


def _build_neg_indices(front, back):
    ok = front < back
    qi = jnp.where(ok, front, 0)
    qj = jnp.where(ok, back, 1)
    q = jnp.where(ok, _tri_offset(qi) + qj - qi - 1, N_PAIRS)
    b_sorted = jnp.sort(q)
    uniq = jnp.concatenate([
        jnp.ones((1,), jnp.bool_),
        b_sorted[1:] != b_sorted[:-1],
    ]) & (b_sorted < N_PAIRS)
    c_pref = jnp.cumsum(uniq.astype(jnp.int32))
    n_neg = N_PAIRS - c_pref[-1]

    p_const = jnp.asarray(_P_CONST)
    flags = p_const < n_neg
    pos = jnp.cumsum(flags.astype(jnp.int32)) - 1
    target = jnp.where(flags & (pos < M), pos, M)
    chosen = jnp.zeros((M + 1,), jnp.int32).at[target].set(
        p_const, mode="drop")[:M]

    rank = c_pref - 1
    du = jnp.full((M,), 1 << 30, jnp.int32).at[
        jnp.where(uniq, rank, M)].set(b_sorted, mode="drop")

    lo = jnp.zeros((M,), jnp.int32)
    hi = jnp.full((M,), M, jnp.int32)

    def bs_body(_, carry):
        lo, hi = carry
        mid = (lo + hi) // 2
        cond = du[jnp.minimum(mid, M - 1)] > chosen + mid
        return jnp.where(cond, lo, mid + 1), jnp.where(cond, mid, hi)

    lo, hi = lax.fori_loop(0, 16, bs_body, (lo, hi))
    pair_idx = chosen + hi

    lo = jnp.zeros((M,), jnp.int32)
    hi = jnp.full((M,), N - 2, jnp.int32)

    def row_body(_, carry):
        lo, hi = carry
        mid = (lo + hi + 1) // 2
        cond = _tri_offset(mid) <= pair_idx
        return jnp.where(cond, mid, lo), jnp.where(cond, hi, mid - 1)

    lo, hi = lax.fori_loop(0, 12, row_body, (lo, hi))
    neg_i = lo
    neg_j = pair_idx - _tri_offset(neg_i) + neg_i + 1
    return neg_i, neg_j


def kernel(x, edge_index):
    front = edge_index[0, ::2]
    back = edge_index[1, ::2]
    neg_i, neg_j = _build_neg_indices(front, back)
    ia = jnp.concatenate([front, neg_i])
    ib = jnp.concatenate([back, neg_j])
    scores = _sc_scores(x, ia, ib)
    return _tc_loss(scores)


# speedup vs baseline: 2.4140x; 2.4140x over previous
import random

import numpy as np
import jax
import jax.numpy as jnp
from jax import lax
from jax.experimental import pallas as pl
from jax.experimental.pallas import tpu as pltpu
from jax.experimental.pallas import tpu_sc as plsc

N = 2000
N_PAIRS = N * (N - 1) // 2
M = 32000
E = 2 * M
N_NEG_MIN = N_PAIRS - M

_rng = random.Random(0)
_pool_np = np.array([_rng.getrandbits(32) >> 11 for _ in range(131072)],
                    dtype=np.int64)
_seen = set()
_u_list = []
for _v in _pool_np:
    _v = int(_v)
    if _v in _seen:
        continue
    _seen.add(_v)
    if _v < N_PAIRS:
        _u_list.append(_v)
_U = np.array(_u_list, dtype=np.int32)
_T_MAX = int(np.searchsorted(np.cumsum(_U < N_NEG_MIN), M, side="left")) + 1
_P_CONST = _U[:_T_MAX]

try:
    _info = plsc.get_sparse_core_info()
    _NC, _NS = _info.num_cores, _info.num_subcores
except Exception:
    _NC, _NS = 2, 16
NW = _NC * _NS
EPW = E // NW
CB = 400
D = 128


def _sc_score_kernel(x_hbm, ia_hbm, ib_hbm, out_hbm,
                     ia_v, ib_v, rows_a, rows_b, scores_v, sem):
    wid = lax.axis_index("s") * _NC + lax.axis_index("c")
    lane = lax.iota(jnp.int32, 16)

    def chunk_body(c, carry):
        base = wid * EPW + c * CB
        pltpu.sync_copy(ia_hbm.at[pl.ds(base, CB)], ia_v)
        pltpu.sync_copy(ib_hbm.at[pl.ds(base, CB)], ib_v)
        pltpu.async_copy(x_hbm.at[ia_v], rows_a, sem).wait()
        pltpu.async_copy(x_hbm.at[ib_v], rows_b, sem).wait()
        def group_body(g, carry2):
            def lane_body(r, vec):
                e = g * 16 + r

                def dim_body(k, acc):
                    a = rows_a[e, pl.ds(k * 16, 16)]
                    b = rows_b[e, pl.ds(k * 16, 16)]
                    return acc + a * b

                acc = lax.fori_loop(0, D // 16, dim_body,
                                    jnp.zeros((16,), jnp.float32))
                s = jnp.sum(acc)
                return jnp.where(lane == r, s, vec)

            vec = lax.fori_loop(0, 16, lane_body, jnp.zeros((16,), jnp.float32))
            scores_v[pl.ds(g * 16, 16)] = vec
            return carry2

        lax.fori_loop(0, CB // 16, group_body, 0)
        pltpu.sync_copy(scores_v, out_hbm.at[pl.ds(base, CB)])
        return carry

    lax.fori_loop(0, EPW // CB, chunk_body, 0)


def _sc_scores(x, ia, ib):
    mesh = plsc.VectorSubcoreMesh(core_axis_name="c", subcore_axis_name="s")
    return pl.kernel(
        _sc_score_kernel,
        mesh=mesh,
        compiler_params=pltpu.CompilerParams(needs_layout_passes=False),
        out_type=jax.ShapeDtypeStruct((E,), jnp.float32),
        scratch_types=[
            pltpu.VMEM((CB,), jnp.int32),
            pltpu.VMEM((CB,), jnp.int32),
            pltpu.VMEM((CB, D), jnp.float32),
            pltpu.VMEM((CB, D), jnp.float32),
            pltpu.VMEM((CB,), jnp.float32),
            pltpu.SemaphoreType.DMA,
        ],
    )(x, ia, ib)


QPW = M // NW


def _sc_neg_idx_kernel(du_hbm, ch_hbm, ni_hbm, nj_hbm,
                       du_v, ch_v, ni_v, nj_v):
    wid = lax.axis_index("s") * _NC + lax.axis_index("c")
    pltpu.sync_copy(du_hbm, du_v)
    pltpu.sync_copy(ch_hbm.at[pl.ds(wid * QPW, QPW)], ch_v)
    n_groups = (QPW + 15) // 16

    def group_body(g, carry):
        start = jnp.minimum(g * 16, QPW - 16)
        ch = ch_v[pl.ds(start, 16)]

        def bs_body(_, c2):
            lo, hi = c2
            mid = (lo + hi) // 2
            v = plsc.load_gather(du_v, [mid])
            cond = v > ch + mid
            return jnp.where(cond, lo, mid + 1), jnp.where(cond, mid, hi)

        lo0 = jnp.zeros((16,), jnp.int32)
        hi0 = jnp.full((16,), M, jnp.int32)
        _, kstar = lax.fori_loop(0, 16, bs_body, (lo0, hi0))
        pair = ch + kstar

        def row_body(_, c2):
            lo, hi = c2
            mid = (lo + hi + 1) // 2
            off = mid * (N - 1) - (mid * (mid - 1)) // 2
            cond = off <= pair
            return jnp.where(cond, mid, lo), jnp.where(cond, hi, mid - 1)

        lo0 = jnp.zeros((16,), jnp.int32)
        hi0 = jnp.full((16,), N - 2, jnp.int32)
        ni, _ = lax.fori_loop(0, 11, row_body, (lo0, hi0))
        nj = pair - (ni * (N - 1) - (ni * (ni - 1)) // 2) + ni + 1
        ni_v[pl.ds(start, 16)] = ni
        nj_v[pl.ds(start, 16)] = nj
        return carry

    lax.fori_loop(0, n_groups, group_body, 0)
    pltpu.sync_copy(ni_v, ni_hbm.at[pl.ds(wid * QPW, QPW)])
    pltpu.sync_copy(nj_v, nj_hbm.at[pl.ds(wid * QPW, QPW)])


def _sc_neg_indices(du, chosen):
    mesh = plsc.VectorSubcoreMesh(core_axis_name="c", subcore_axis_name="s")
    return pl.kernel(
        _sc_neg_idx_kernel,
        mesh=mesh,
        compiler_params=pltpu.CompilerParams(needs_layout_passes=False),
        out_type=(jax.ShapeDtypeStruct((M,), jnp.int32),
                  jax.ShapeDtypeStruct((M,), jnp.int32)),
        scratch_types=[
            pltpu.VMEM((M,), jnp.int32),
            pltpu.VMEM((QPW,), jnp.int32),
            pltpu.VMEM((QPW,), jnp.int32),
            pltpu.VMEM((QPW,), jnp.int32),
        ],
    )(du, chosen)


def _tc_loss_kernel(s_ref, o_ref):
    s = s_ref[...]
    row = lax.broadcasted_iota(jnp.int32, s.shape, 0)
    z = jnp.where(row < (M // 128), -s, s)
    sp = jnp.maximum(z, 0.0) + jnp.log1p(jnp.exp(-jnp.abs(z)))
    total = jnp.sum(sp) * (1.0 / (float(M) * float(M)))
    o_ref[...] = jnp.broadcast_to(total, (1, 1))


def _tc_loss(scores):
    s2 = scores.reshape(E // 128, 128)
    out = pl.pallas_call(
        _tc_loss_kernel,
        out_shape=jax.ShapeDtypeStruct((1, 1), jnp.float32),
    )(s2)
    return out[0, 0]


def _tri_offset(i):
    return i * (N - 1) - (i * (i - 1)) // 2


def _build_neg_indices(front, back):
    ok = front < back
    qi = jnp.where(ok, front, 0)
    qj = jnp.where(ok, back, 1)
    q = jnp.where(ok, _tri_offset(qi) + qj - qi - 1, N_PAIRS)
    b_sorted = jnp.sort(q)
    uniq = jnp.concatenate([
        jnp.ones((1,), jnp.bool_),
        b_sorted[1:] != b_sorted[:-1],
    ]) & (b_sorted < N_PAIRS)
    c_pref = jnp.cumsum(uniq.astype(jnp.int32))
    n_neg = N_PAIRS - c_pref[-1]

    p_const = jnp.asarray(_P_CONST)
    flags = p_const < n_neg
    pos = jnp.cumsum(flags.astype(jnp.int32)) - 1
    target = jnp.where(flags & (pos < M), pos, M)
    chosen = jnp.zeros((M + 1,), jnp.int32).at[target].set(
        p_const, mode="drop")[:M]

    rank = c_pref - 1
    du = jnp.full((M,), 1 << 30, jnp.int32).at[
        jnp.where(uniq, rank, M)].set(b_sorted, mode="drop")

    return du, chosen


def kernel(x, edge_index):
    front = edge_index[0, ::2]
    back = edge_index[1, ::2]
    du, chosen = _build_neg_indices(front, back)
    neg_i, neg_j = _sc_neg_indices(du, chosen)
    ia = jnp.concatenate([front, neg_i])
    ib = jnp.concatenate([back, neg_j])
    scores = _sc_scores(x, ia, ib)
    return _tc_loss(scores)


# speedup vs baseline: 2.4353x; 1.0088x over previous
import random

import numpy as np
import jax
import jax.numpy as jnp
from jax import lax
from jax.experimental import pallas as pl
from jax.experimental.pallas import tpu as pltpu
from jax.experimental.pallas import tpu_sc as plsc

N = 2000
N_PAIRS = N * (N - 1) // 2
M = 32000
E = 2 * M
N_NEG_MIN = N_PAIRS - M

_rng = random.Random(0)
_pool_np = np.array([_rng.getrandbits(32) >> 11 for _ in range(131072)],
                    dtype=np.int64)
_seen = set()
_u_list = []
for _v in _pool_np:
    _v = int(_v)
    if _v in _seen:
        continue
    _seen.add(_v)
    if _v < N_PAIRS:
        _u_list.append(_v)
_U = np.array(_u_list, dtype=np.int32)
_T_MAX = int(np.searchsorted(np.cumsum(_U < N_NEG_MIN), M, side="left")) + 1
_P_CONST = _U[:_T_MAX]

try:
    _info = plsc.get_sparse_core_info()
    _NC, _NS = _info.num_cores, _info.num_subcores
except Exception:
    _NC, _NS = 2, 16
NW = _NC * _NS
QPW = M // NW
CB = 200
D = 128


def _sc_main_kernel(x_hbm, fr_hbm, bk_hbm, du_hbm, ch_hbm, out_hbm,
                    du_v, ch_v, ni_v, nj_v, ia_v, ib_v,
                    rows_a, rows_b, scores_v, sem):
    wid = lax.axis_index("s") * _NC + lax.axis_index("c")
    lane = lax.iota(jnp.int32, 16)

    pltpu.sync_copy(du_hbm, du_v)
    pltpu.sync_copy(ch_hbm.at[pl.ds(wid * QPW, QPW)], ch_v)
    pltpu.sync_copy(fr_hbm.at[pl.ds(wid * QPW, QPW)], ia_v)
    pltpu.sync_copy(bk_hbm.at[pl.ds(wid * QPW, QPW)], ib_v)

    def group_body(g, carry):
        start = jnp.minimum(g * 16, QPW - 16)
        ch = ch_v[pl.ds(start, 16)]

        def bs_body(_, c2):
            lo, hi = c2
            mid = (lo + hi) // 2
            v = plsc.load_gather(du_v, [mid])
            cond = v > ch + mid
            return jnp.where(cond, lo, mid + 1), jnp.where(cond, mid, hi)

        lo0 = jnp.zeros((16,), jnp.int32)
        hi0 = jnp.full((16,), M, jnp.int32)
        _, kstar = lax.fori_loop(0, 16, bs_body, (lo0, hi0))
        pair = ch + kstar

        def row_body(_, c2):
            lo, hi = c2
            mid = (lo + hi + 1) // 2
            off = mid * (N - 1) - (mid * (mid - 1)) // 2
            cond = off <= pair
            return jnp.where(cond, mid, lo), jnp.where(cond, hi, mid - 1)

        lo0 = jnp.zeros((16,), jnp.int32)
        hi0 = jnp.full((16,), N - 2, jnp.int32)
        ni, _ = lax.fori_loop(0, 11, row_body, (lo0, hi0))
        nj = pair - (ni * (N - 1) - (ni * (ni - 1)) // 2) + ni + 1
        ni_v[pl.ds(start, 16)] = ni
        nj_v[pl.ds(start, 16)] = nj
        return carry

    lax.fori_loop(0, (QPW + 15) // 16, group_body, 0)

    def score_chunk(a_idx, b_idx, a_off, b_off, out_base):
        pltpu.async_copy(x_hbm.at[a_idx.at[pl.ds(a_off, CB)]], rows_a,
                         sem).wait()
        pltpu.async_copy(x_hbm.at[b_idx.at[pl.ds(b_off, CB)]], rows_b,
                         sem).wait()

        def cgroup_body(g, carry2):
            start = jnp.minimum(g * 16, CB - 16)

            def lane_body(r, vec):
                e = start + r

                def dim_body(k, acc):
                    a = rows_a[e, pl.ds(k * 16, 16)]
                    b = rows_b[e, pl.ds(k * 16, 16)]
                    return acc + a * b

                acc = lax.fori_loop(0, D // 16, dim_body,
                                    jnp.zeros((16,), jnp.float32))
                s = jnp.sum(acc)
                return jnp.where(lane == r, s, vec)

            vec = lax.fori_loop(0, 16, lane_body,
                                jnp.zeros((16,), jnp.float32))
            scores_v[pl.ds(start, 16)] = vec
            return carry2

        lax.fori_loop(0, (CB + 15) // 16, cgroup_body, 0)
        pltpu.sync_copy(scores_v, out_hbm.at[pl.ds(out_base, CB)])

    def pos_chunk(c, carry):
        score_chunk(ia_v, ib_v, c * CB, c * CB, wid * QPW + c * CB)
        return carry

    lax.fori_loop(0, QPW // CB, pos_chunk, 0)

    def neg_chunk(c, carry):
        score_chunk(ni_v, nj_v, c * CB, c * CB, M + wid * QPW + c * CB)
        return carry

    lax.fori_loop(0, QPW // CB, neg_chunk, 0)


def _sc_scores(x, front, back, du, chosen):
    mesh = plsc.VectorSubcoreMesh(core_axis_name="c", subcore_axis_name="s")
    return pl.kernel(
        _sc_main_kernel,
        mesh=mesh,
        compiler_params=pltpu.CompilerParams(needs_layout_passes=False),
        out_type=jax.ShapeDtypeStruct((E,), jnp.float32),
        scratch_types=[
            pltpu.VMEM((M,), jnp.int32),
            pltpu.VMEM((QPW,), jnp.int32),
            pltpu.VMEM((QPW,), jnp.int32),
            pltpu.VMEM((QPW,), jnp.int32),
            pltpu.VMEM((QPW,), jnp.int32),
            pltpu.VMEM((QPW,), jnp.int32),
            pltpu.VMEM((CB, D), jnp.float32),
            pltpu.VMEM((CB, D), jnp.float32),
            pltpu.VMEM((CB,), jnp.float32),
            pltpu.SemaphoreType.DMA,
        ],
    )(x, front, back, du, chosen)


def _tc_loss_kernel(s_ref, o_ref):
    s = s_ref[...]
    row = lax.broadcasted_iota(jnp.int32, s.shape, 0)
    z = jnp.where(row < (M // 128), -s, s)
    sp = jnp.maximum(z, 0.0) + jnp.log1p(jnp.exp(-jnp.abs(z)))
    total = jnp.sum(sp) * (1.0 / (float(M) * float(M)))
    o_ref[...] = jnp.broadcast_to(total, (1, 1))


def _tc_loss(scores):
    s2 = scores.reshape(E // 128, 128)
    out = pl.pallas_call(
        _tc_loss_kernel,
        out_shape=jax.ShapeDtypeStruct((1, 1), jnp.float32),
    )(s2)
    return out[0, 0]


def _tri_offset(i):
    return i * (N - 1) - (i * (i - 1)) // 2


def _build_neg_indices(front, back):
    ok = front < back
    qi = jnp.where(ok, front, 0)
    qj = jnp.where(ok, back, 1)
    q = jnp.where(ok, _tri_offset(qi) + qj - qi - 1, N_PAIRS)
    b_sorted = jnp.sort(q)
    uniq = jnp.concatenate([
        jnp.ones((1,), jnp.bool_),
        b_sorted[1:] != b_sorted[:-1],
    ]) & (b_sorted < N_PAIRS)
    c_pref = jnp.cumsum(uniq.astype(jnp.int32))
    n_neg = N_PAIRS - c_pref[-1]

    p_const = jnp.asarray(_P_CONST)
    flags = p_const < n_neg
    pos = jnp.cumsum(flags.astype(jnp.int32)) - 1
    target = jnp.where(flags & (pos < M), pos, M)
    chosen = jnp.zeros((M + 1,), jnp.int32).at[target].set(
        p_const, mode="drop")[:M]

    rank = c_pref - 1
    du = jnp.full((M,), 1 << 30, jnp.int32).at[
        jnp.where(uniq, rank, M)].set(b_sorted, mode="drop")

    return du, chosen


def kernel(x, edge_index):
    front = edge_index[0, ::2]
    back = edge_index[1, ::2]
    du, chosen = _build_neg_indices(front, back)
    scores = _sc_scores(x, front, back, du, chosen)
    return _tc_loss(scores)
